# baseline (speedup 1.0000x reference)
import functools

import jax
import jax.numpy as jnp
from jax import lax
from jax.experimental import pallas as pl
from jax.experimental.pallas import tpu as pltpu
from jax.experimental.pallas import tpu_sc as plsc

_BATCH = 16384
_D = 64
_NC = 2
_NS = 16
_NW = _NC * _NS
_PER_W = _BATCH // _NW
_C = 64
_NCHUNK = _PER_W // _C
_NG = _C // 16
_MARGIN = 1.0
_PACK_W = 32768


def _pack_body(x, o):
    ident = jnp.eye(128, dtype=jnp.float32)
    ones = jnp.ones((1, _D), dtype=jnp.float32)
    red = (((1,), (0,)), ((), ()))
    dn = (((0,), (0,)), ((), ()))
    for u in range(x.shape[1] // 256):
        a = x[:, u * 256:u * 256 + 128]
        b = x[:, u * 256 + 128:u * 256 + 256]
        inva = lax.rsqrt(jnp.maximum(
            lax.dot_general(ones, a * a, red,
                            preferred_element_type=jnp.float32), 1e-12))
        invb = lax.rsqrt(jnp.maximum(
            lax.dot_general(ones, b * b, red,
                            preferred_element_type=jnp.float32), 1e-12))
        stacked = jnp.concatenate([a * inva, b * invb], axis=0)
        o[u * 128:(u + 1) * 128, :] = lax.dot_general(
            stacked, ident, dn, preferred_element_type=jnp.float32)


def _pack(table_t, w):
    nblk = (table_t.shape[1] + w - 1) // w
    return pl.pallas_call(
        _pack_body,
        grid=(nblk,),
        in_specs=[pl.BlockSpec((_D, w), lambda j: (0, j))],
        out_specs=pl.BlockSpec((w // 2, 128), lambda j: (j, 0)),
        out_shape=jax.ShapeDtypeStruct((nblk * w // 2, 128), jnp.float32),
    )(table_t)


def _row_of(i):
    return jnp.left_shift(jnp.right_shift(i, 8), 7) + jnp.bitwise_and(i, 127)


def _off_of(i):
    return jnp.left_shift(jnp.bitwise_and(jnp.right_shift(i, 7), 1), 6)


def _body(pos_h, pos_t, pos_r, neg_h, neg_t, neg_r, ent_p, rel_p, out,
          i_ph, i_pt, i_pr, i_nh, i_nt, i_nr,
          a_e1, a_e2, a_r,
          bA_e1, bA_e2, bA_r, bB_e1, bB_e2, bB_r,
          acc_v, semA, semB):
    wid = lax.axis_index("s") * _NC + lax.axis_index("c")
    iota = lax.iota(jnp.int32, 16)
    raws = (i_ph, i_pt, i_pr, i_nh, i_nt, i_nr)
    bufs = ((bA_e1, bA_e2, bA_r), (bB_e1, bB_e2, bB_r))
    sems = (semA, semB)

    idx_cps = [
        pltpu.async_copy(src.at[pl.ds(wid * _PER_W, _PER_W)], raw_ref, semA)
        for src, raw_ref in zip((pos_h, pos_t, pos_r, neg_h, neg_t, neg_r),
                                raws)
    ]
    for cp in idx_cps:
        cp.wait()

    def remap2(x_ref, y_ref, dst_ref):
        def step(v, _):
            base = jnp.left_shift(jnp.right_shift(v, 2), 7) \
                + jnp.left_shift(jnp.bitwise_and(v, 3), 4)
            dst_ref[pl.ds(base, 16)] = _row_of(x_ref[pl.ds(v * 16, 16)])
            dst_ref[pl.ds(base + 64, 16)] = _row_of(y_ref[pl.ds(v * 16, 16)])
            return 0

        lax.fori_loop(0, _PER_W // 16, step, 0)

    remap2(i_ph, i_pt, a_e1)
    remap2(i_nh, i_nt, a_e2)
    remap2(i_pr, i_nr, a_r)

    def fire(k):
        sl = pl.ds(k * 128, 128)
        b = bufs[k % 2]
        sem = sems[k % 2]
        return [
            pltpu.async_copy(ent_p.at[a_e1.at[sl]], b[0], sem),
            pltpu.async_copy(ent_p.at[a_e2.at[sl]], b[1], sem),
            pltpu.async_copy(rel_p.at[a_r.at[sl]], b[2], sem),
        ]

    def make_score_step(b):
        def score_step(d, carry):
            pacc, nacc, row, offs = carry
            h = plsc.load_gather(b[0], [row, offs[0] + d])
            t = plsc.load_gather(b[0], [row + 64, offs[1] + d])
            r = plsc.load_gather(b[2], [row, offs[2] + d])
            pacc = pacc + jnp.abs(h + r - t)
            h = plsc.load_gather(b[1], [row, offs[3] + d])
            t = plsc.load_gather(b[1], [row + 64, offs[4] + d])
            r = plsc.load_gather(b[2], [row + 64, offs[5] + d])
            nacc = nacc + jnp.abs(h + r - t)
            return pacc, nacc, row, offs

        return score_step

    def compute(k, total):
        b = bufs[k % 2]
        step = make_score_step(b)

        def group_body(g, tot):
            row = g * 16 + iota
            offs = tuple(_off_of(raw_ref[pl.ds(k * _C + g * 16, 16)])
                         for raw_ref in raws)
            z = jnp.zeros((16,), jnp.float32)
            pacc, nacc, _, _ = lax.fori_loop(
                0, _D, step, (z, z, row, offs), unroll=8)
            return tot + jnp.maximum(pacc - nacc + _MARGIN, 0.0)

        return lax.fori_loop(0, _NG, group_body, total)

    total = jnp.zeros((16,), jnp.float32)
    cps = fire(0)
    for k in range(_NCHUNK):
        nxt = fire(k + 1) if k + 1 < _NCHUNK else []
        for cp in cps:
            cp.wait()
        total = compute(k, total)
        cps = nxt

    acc_v[...] = total * (1.0 / _BATCH)
    pltpu.sync_copy(acc_v, out.at[wid])


@jax.jit
def kernel(pos_h, pos_t, pos_r, neg_h, neg_t, neg_r, ent_emb, rel_emb):
    ent_p = _pack(jnp.transpose(ent_emb), _PACK_W)
    rel_p = _pack(jnp.transpose(rel_emb), 1024)
    mesh = plsc.VectorSubcoreMesh(core_axis_name="c", subcore_axis_name="s",
                                  num_cores=_NC, num_subcores=_NS)
    run = functools.partial(
        pl.kernel,
        out_type=jax.ShapeDtypeStruct((_NW, 16), jnp.float32),
        mesh=mesh,
        scratch_types=[pltpu.VMEM((_PER_W,), jnp.int32)] * 6
        + [pltpu.VMEM((2 * _PER_W,), jnp.int32)] * 3
        + [pltpu.VMEM((128, 128), jnp.float32)] * 6
        + [pltpu.VMEM((16,), jnp.float32),
           pltpu.SemaphoreType.DMA, pltpu.SemaphoreType.DMA],
        compiler_params=pltpu.CompilerParams(needs_layout_passes=False,
                                             use_tc_tiling_on_sc=False),
    )(_body)
    partial_sums = run(pos_h, pos_t, pos_r, neg_h, neg_t, neg_r,
                       ent_p, rel_p)
    return jnp.sum(partial_sums)


# speedup vs baseline: 1.0081x; 1.0081x over previous
import functools

import jax
import jax.numpy as jnp
from jax import lax
from jax.experimental import pallas as pl
from jax.experimental.pallas import tpu as pltpu
from jax.experimental.pallas import tpu_sc as plsc

_BATCH = 16384
_D = 64
_NC = 2
_NS = 16
_NW = _NC * _NS
_PER_W = _BATCH // _NW
_C = 64
_NCHUNK = _PER_W // _C
_NG = _C // 16
_MARGIN = 1.0
_PACK_W = 32768


def _pack_body(x, o):
    ident = jnp.eye(128, dtype=jnp.float32)
    ones = jnp.ones((1, _D), dtype=jnp.float32)
    red = (((1,), (0,)), ((), ()))
    dn = (((0,), (0,)), ((), ()))
    for u in range(x.shape[1] // 256):
        a = x[:, u * 256:u * 256 + 128]
        b = x[:, u * 256 + 128:u * 256 + 256]
        inva = lax.rsqrt(jnp.maximum(
            lax.dot_general(ones, a * a, red,
                            preferred_element_type=jnp.float32), 1e-12))
        invb = lax.rsqrt(jnp.maximum(
            lax.dot_general(ones, b * b, red,
                            preferred_element_type=jnp.float32), 1e-12))
        stacked = jnp.concatenate([a * inva, b * invb], axis=0)
        o[u * 128:(u + 1) * 128, :] = lax.dot_general(
            stacked, ident, dn, preferred_element_type=jnp.float32)


def _pack(table_t, w):
    nblk = (table_t.shape[1] + w - 1) // w
    return pl.pallas_call(
        _pack_body,
        grid=(nblk,),
        in_specs=[pl.BlockSpec((_D, w), lambda j: (0, j))],
        out_specs=pl.BlockSpec((w // 2, 128), lambda j: (j, 0)),
        out_shape=jax.ShapeDtypeStruct((nblk * w // 2, 128), jnp.float32),
    )(table_t)


def _row_of(i):
    return jnp.left_shift(jnp.right_shift(i, 8), 7) + jnp.bitwise_and(i, 127)


def _off_of(i):
    return jnp.left_shift(jnp.bitwise_and(jnp.right_shift(i, 7), 1), 6)


def _body(pos_h, pos_t, pos_r, neg_h, neg_t, neg_r, ent_p, rel_p, out,
          i_ph, i_pt, i_pr, i_nh, i_nt, i_nr,
          a_ph, a_pt, a_pr, a_nh, a_nt, a_nr,
          rA_ph, rA_pt, rA_pr, rA_nh, rA_nt, rA_nr,
          rB_ph, rB_pt, rB_pr, rB_nh, rB_nt, rB_nr,
          acc_v, semA, semB):
    wid = lax.axis_index("s") * _NC + lax.axis_index("c")
    iota = lax.iota(jnp.int32, 16)
    raws = (i_ph, i_pt, i_pr, i_nh, i_nt, i_nr)
    adjs = (a_ph, a_pt, a_pr, a_nh, a_nt, a_nr)
    bufs = ((rA_ph, rA_pt, rA_pr, rA_nh, rA_nt, rA_nr),
            (rB_ph, rB_pt, rB_pr, rB_nh, rB_nt, rB_nr))
    sems = (semA, semB)

    idx_cps = [
        pltpu.async_copy(src.at[pl.ds(wid * _PER_W, _PER_W)], raw_ref, semA)
        for src, raw_ref in zip((pos_h, pos_t, pos_r, neg_h, neg_t, neg_r),
                                raws)
    ]
    for cp in idx_cps:
        cp.wait()

    def remap(raw_ref, adj_ref):
        def step(v, _):
            x = raw_ref[pl.ds(v * 16, 16)]
            adj_ref[pl.ds(v * 16, 16)] = _row_of(x)
            return 0

        lax.fori_loop(0, _PER_W // 16, step, 0)

    for raw_ref, adj_ref in zip(raws, adjs):
        remap(raw_ref, adj_ref)

    def fire(k):
        sl = pl.ds(k * _C, _C)
        b = bufs[k % 2]
        sem = sems[k % 2]
        return [
            pltpu.async_copy(ent_p.at[a_ph.at[sl]], b[0], sem),
            pltpu.async_copy(ent_p.at[a_pt.at[sl]], b[1], sem),
            pltpu.async_copy(rel_p.at[a_pr.at[sl]], b[2], sem),
            pltpu.async_copy(ent_p.at[a_nh.at[sl]], b[3], sem),
            pltpu.async_copy(ent_p.at[a_nt.at[sl]], b[4], sem),
            pltpu.async_copy(rel_p.at[a_nr.at[sl]], b[5], sem),
        ]

    def make_score_step(b):
        def score_step(d, carry):
            pacc, nacc, row, offs = carry
            h = plsc.load_gather(b[0], [row, offs[0] + d])
            t = plsc.load_gather(b[1], [row, offs[1] + d])
            r = plsc.load_gather(b[2], [row, offs[2] + d])
            pacc = pacc + jnp.abs(h + r - t)
            h = plsc.load_gather(b[3], [row, offs[3] + d])
            t = plsc.load_gather(b[4], [row, offs[4] + d])
            r = plsc.load_gather(b[5], [row, offs[5] + d])
            nacc = nacc + jnp.abs(h + r - t)
            return pacc, nacc, row, offs

        return score_step

    def compute(k, total):
        b = bufs[k % 2]
        step = make_score_step(b)

        def group_body(g, tot):
            row = g * 16 + iota
            offs = tuple(_off_of(raw_ref[pl.ds(k * _C + g * 16, 16)])
                         for raw_ref in raws)
            z = jnp.zeros((16,), jnp.float32)
            pacc, nacc, _, _ = lax.fori_loop(
                0, _D, step, (z, z, row, offs), unroll=8)
            return tot + jnp.maximum(pacc - nacc + _MARGIN, 0.0)

        return lax.fori_loop(0, _NG, group_body, total)

    total = jnp.zeros((16,), jnp.float32)
    cps = fire(0)
    for k in range(_NCHUNK):
        nxt = fire(k + 1) if k + 1 < _NCHUNK else []
        for cp in cps:
            cp.wait()
        total = compute(k, total)
        cps = nxt

    acc_v[...] = total * (1.0 / _BATCH)
    pltpu.sync_copy(acc_v, out.at[wid])


@jax.jit
def kernel(pos_h, pos_t, pos_r, neg_h, neg_t, neg_r, ent_emb, rel_emb):
    ent_p = _pack(jnp.transpose(ent_emb), _PACK_W)
    rel_p = _pack(jnp.transpose(rel_emb), 1024)
    mesh = plsc.VectorSubcoreMesh(core_axis_name="c", subcore_axis_name="s",
                                  num_cores=_NC, num_subcores=_NS)
    run = functools.partial(
        pl.kernel,
        out_type=jax.ShapeDtypeStruct((_NW, 16), jnp.float32),
        mesh=mesh,
        scratch_types=[pltpu.VMEM((_PER_W,), jnp.int32)] * 12
        + [pltpu.VMEM((_C, 128), jnp.float32)] * 12
        + [pltpu.VMEM((16,), jnp.float32),
           pltpu.SemaphoreType.DMA, pltpu.SemaphoreType.DMA],
        compiler_params=pltpu.CompilerParams(needs_layout_passes=False,
                                             use_tc_tiling_on_sc=False),
    )(_body)
    partial_sums = run(pos_h, pos_t, pos_r, neg_h, neg_t, neg_r,
                       ent_p, rel_p)
    return jnp.sum(partial_sums)
